# baseline (speedup 1.0000x reference)
import functools

import jax
import jax.numpy as jnp
from jax.experimental import pallas as pl
from jax.experimental.pallas import tpu as pltpu

B, L, D, D_OUT = 16, 2048, 1024, 1024
NS = 4
LC = L // NS


def _chunk(x_ref, w1_ref, v_ref, mask, lo):
    xb = x_ref[0, 0].astype(jnp.bfloat16)
    h = jnp.tanh(
        jax.lax.dot_general(xb, w1_ref[...], (((1,), (0,)), ((), ())),
                            preferred_element_type=jnp.float32))
    s = jnp.sum(h * v_ref[...], axis=1, keepdims=True)
    s = jnp.where(mask[lo:lo + LC] > 0, s, jnp.float32(-1e9))
    return xb, s


def _body(x0_ref, x1_ref, x2_ref, x3_ref, mask_ref, w1_ref, v_ref, wr_ref,
          br_ref, out_ref, pool_ref):
    b = pl.program_id(0)
    mask = mask_ref[0]

    xrefs = (x0_ref, x1_ref, x2_ref, x3_ref)
    xbs, ss = [], []
    for i, xr in enumerate(xrefs):
        xb, s = _chunk(xr, w1_ref, v_ref, mask, i * LC)
        xbs.append(xb)
        ss.append(s)

    scores = jnp.concatenate(ss, axis=0)
    m = jnp.max(scores)
    p = jnp.exp(scores - m)
    pw = (p / jnp.sum(p)).astype(jnp.bfloat16)
    pooled = sum(
        jax.lax.dot_general(pw[i * LC:(i + 1) * LC], xbs[i],
                            (((0,), (0,)), ((), ())),
                            preferred_element_type=jnp.float32)
        for i in range(NS))
    pool_ref[pl.ds(b, 1), :] = pooled

    @pl.when(b == B - 1)
    def _finish():
        out_ref[...] = jax.lax.dot_general(
            pool_ref[...], wr_ref[...], (((1,), (0,)), ((), ())),
            preferred_element_type=jnp.float32) + br_ref[...]


@functools.partial(jax.jit, static_argnames=())
def kernel(word_hidden, word_mask, W1, v, Wr, br):
    maskf = word_mask.astype(jnp.float32).reshape(B, L, 1)
    xs = word_hidden.reshape(B, NS, LC, D)
    w1_bf = W1.astype(jnp.bfloat16)
    v2 = v.reshape(1, D)
    br2 = br.reshape(1, D_OUT)

    def xspec(i):
        return pl.BlockSpec((1, 1, LC, D), lambda b, i=i: (b, i, 0, 0))

    out = pl.pallas_call(
        _body,
        grid=(B,),
        in_specs=[
            xspec(0), xspec(1), xspec(2), xspec(3),
            pl.BlockSpec((1, L, 1), lambda b: (b, 0, 0)),
            pl.BlockSpec((D, D), lambda b: (0, 0)),
            pl.BlockSpec((1, D), lambda b: (0, 0)),
            pl.BlockSpec((D, D_OUT), lambda b: (0, 0)),
            pl.BlockSpec((1, D_OUT), lambda b: (0, 0)),
        ],
        out_specs=pl.BlockSpec((B, D_OUT), lambda b: (0, 0)),
        out_shape=jax.ShapeDtypeStruct((B, D_OUT), jnp.float32),
        scratch_shapes=[
            pltpu.VMEM((B, D), jnp.float32),
        ],
        compiler_params=pltpu.CompilerParams(
            dimension_semantics=("arbitrary",)),
    )(xs, xs, xs, xs, maskf, w1_bf, v2, Wr, br2)
    return out


# speedup vs baseline: 1.0749x; 1.0749x over previous
import functools

import jax
import jax.numpy as jnp
from jax.experimental import pallas as pl
from jax.experimental.pallas import tpu as pltpu

B, L, D, D_OUT = 16, 2048, 1024, 1024
L2 = L // 2


def _half(x_ref, w1_ref, v_ref, mask, lo):
    xb = x_ref[0, 0].astype(jnp.bfloat16)
    h = jnp.tanh(
        jax.lax.dot_general(xb, w1_ref[...], (((1,), (0,)), ((), ())),
                            preferred_element_type=jnp.float32))
    s = jnp.sum(h * v_ref[...], axis=1, keepdims=True)
    p = jnp.exp(s) * mask[lo:lo + L2]
    return xb, p


def _body(x0_ref, x1_ref, mask_ref, w1_ref, v_ref, wr_ref, br_ref, out_ref,
          pool_ref):
    b = pl.program_id(0)
    mask = mask_ref[0]

    xb0, p0 = _half(x0_ref, w1_ref, v_ref, mask, 0)
    xb1, p1 = _half(x1_ref, w1_ref, v_ref, mask, L2)

    denom = jnp.sum(p0) + jnp.sum(p1)
    pooled_u = (
        jax.lax.dot_general(p0.astype(jnp.bfloat16), xb0,
                            (((0,), (0,)), ((), ())),
                            preferred_element_type=jnp.float32)
        + jax.lax.dot_general(p1.astype(jnp.bfloat16), xb1,
                              (((0,), (0,)), ((), ())),
                              preferred_element_type=jnp.float32))
    pool_ref[pl.ds(b, 1), :] = pooled_u * (1.0 / denom)

    @pl.when(b == B - 1)
    def _finish():
        out_ref[...] = jax.lax.dot_general(
            pool_ref[...], wr_ref[...], (((1,), (0,)), ((), ())),
            preferred_element_type=jnp.float32) + br_ref[...]


@functools.partial(jax.jit, static_argnames=())
def kernel(word_hidden, word_mask, W1, v, Wr, br):
    maskf = word_mask.astype(jnp.float32).reshape(B, L, 1)
    xs = word_hidden.reshape(B, 2, L2, D)
    w1_bf = W1.astype(jnp.bfloat16)
    v2 = v.reshape(1, D)
    br2 = br.reshape(1, D_OUT)
    out = pl.pallas_call(
        _body,
        grid=(B,),
        in_specs=[
            pl.BlockSpec((1, 1, L2, D), lambda b: (b, 0, 0, 0)),
            pl.BlockSpec((1, 1, L2, D), lambda b: (b, 1, 0, 0)),
            pl.BlockSpec((1, L, 1), lambda b: (b, 0, 0)),
            pl.BlockSpec((D, D), lambda b: (0, 0)),
            pl.BlockSpec((1, D), lambda b: (0, 0)),
            pl.BlockSpec((D, D_OUT), lambda b: (0, 0)),
            pl.BlockSpec((1, D_OUT), lambda b: (0, 0)),
        ],
        out_specs=pl.BlockSpec((B, D_OUT), lambda b: (0, 0)),
        out_shape=jax.ShapeDtypeStruct((B, D_OUT), jnp.float32),
        scratch_shapes=[
            pltpu.VMEM((B, D), jnp.float32),
        ],
        compiler_params=pltpu.CompilerParams(
            dimension_semantics=("arbitrary",)),
    )(xs, xs, maskf, w1_bf, v2, Wr, br2)
    return out


# speedup vs baseline: 1.1212x; 1.0431x over previous
import functools

import jax
import jax.numpy as jnp
from jax.experimental import pallas as pl
from jax.experimental.pallas import tpu as pltpu

B, L, D, D_OUT = 16, 2048, 1024, 1024
NS = 4
LC = L // NS


def _chunk(x_ref, w1_ref, v_ref, mask, lo):
    xb = x_ref[0, 0].astype(jnp.bfloat16)
    h = jnp.tanh(
        jax.lax.dot_general(xb, w1_ref[...], (((1,), (0,)), ((), ())),
                            preferred_element_type=jnp.float32))
    s = jnp.sum(h * v_ref[...], axis=1, keepdims=True)
    p = jnp.exp(s) * mask[lo:lo + LC]
    return xb, p


def _body(x0_ref, x1_ref, x2_ref, x3_ref, mask_ref, w1_ref, v_ref, wr_ref,
          br_ref, out_ref, pool_ref, denom_ref):
    b = pl.program_id(0)
    mask = mask_ref[0]

    xbs, ps = [], []
    for i, xr in enumerate((x0_ref, x1_ref, x2_ref, x3_ref)):
        xb, p = _chunk(xr, w1_ref, v_ref, mask, i * LC)
        xbs.append(xb)
        ps.append(p)

    denom_ref[pl.ds(b, 1), :] = sum(jnp.sum(p) for p in ps).reshape(1, 1)
    pooled_u = sum(
        jax.lax.dot_general(ps[i].astype(jnp.bfloat16), xbs[i],
                            (((0,), (0,)), ((), ())),
                            preferred_element_type=jnp.float32)
        for i in range(NS))
    pool_ref[pl.ds(b, 1), :] = pooled_u

    @pl.when(b == B - 1)
    def _finish():
        pooled = pool_ref[...] / denom_ref[...]
        out_ref[...] = jax.lax.dot_general(
            pooled, wr_ref[...], (((1,), (0,)), ((), ())),
            preferred_element_type=jnp.float32) + br_ref[...]


@functools.partial(jax.jit, static_argnames=())
def kernel(word_hidden, word_mask, W1, v, Wr, br):
    maskf = word_mask.astype(jnp.float32).reshape(B, L, 1)
    xs = word_hidden.reshape(B, NS, LC, D)
    w1_bf = W1.astype(jnp.bfloat16)
    v2 = v.reshape(1, D)
    br2 = br.reshape(1, D_OUT)

    def xspec(i):
        return pl.BlockSpec((1, 1, LC, D), lambda b, i=i: (b, i, 0, 0))

    out = pl.pallas_call(
        _body,
        grid=(B,),
        in_specs=[
            xspec(0), xspec(1), xspec(2), xspec(3),
            pl.BlockSpec((1, L, 1), lambda b: (b, 0, 0)),
            pl.BlockSpec((D, D), lambda b: (0, 0)),
            pl.BlockSpec((1, D), lambda b: (0, 0)),
            pl.BlockSpec((D, D_OUT), lambda b: (0, 0)),
            pl.BlockSpec((1, D_OUT), lambda b: (0, 0)),
        ],
        out_specs=pl.BlockSpec((B, D_OUT), lambda b: (0, 0)),
        out_shape=jax.ShapeDtypeStruct((B, D_OUT), jnp.float32),
        scratch_shapes=[
            pltpu.VMEM((B, D), jnp.float32),
            pltpu.VMEM((B, 1), jnp.float32),
        ],
        compiler_params=pltpu.CompilerParams(
            dimension_semantics=("arbitrary",)),
    )(xs, xs, xs, xs, maskf, w1_bf, v2, Wr, br2)
    return out
